# baseline (speedup 1.0000x reference)
import functools
import jax
import jax.numpy as jnp
from jax import lax
from jax.experimental import pallas as pl
from jax.experimental.pallas import tpu as pltpu
from jax.experimental.pallas import tpu_sc as plsc

N = 10000
E = 320000
L = 100000
H = 128
VOCAB = 1000

NC = 2
NS = 16
NW = NC * NS

EC = 112
EPAD = 322560
ERT = EPAD // (NS * EC)
BR = 4
SBR = 12
SB = ERT // SBR
AGH = 5120
AGR = AGH
ZR = AGR // NS
WBT = AGH // NS
NAGG = NC * AGH

NPAD = 10240
XPT = NPAD // NW
XC = 80
LPAD = 100352
LC = 112
LRT = LPAD // (NW * LC)

_MESH = plsc.VectorSubcoreMesh(
    core_axis_name="c", subcore_axis_name="s", num_cores=NC, num_subcores=NS)
_SC_PARAMS = pltpu.CompilerParams(needs_layout_passes=False)


def _leaky(x):
    return jnp.where(x >= 0, x, 0.01 * x)


def _enc_table_kernel(t_ref, w0_ref, b0_ref, w1_ref, b1_ref, wn0_ref,
                      tx_ref, ty_ref):
    x = t_ref[...]
    x = _leaky(jnp.dot(x, w0_ref[...], preferred_element_type=jnp.float32) + b0_ref[...])
    x = _leaky(jnp.dot(x, w1_ref[...], preferred_element_type=jnp.float32) + b1_ref[...])
    tx_ref[...] = x
    ty_ref[...] = jnp.dot(x, wn0_ref[...], preferred_element_type=jnp.float32)


def _enc_table(t, w0, b0, w1, b1, wn0):
    return pl.pallas_call(
        _enc_table_kernel,
        out_shape=(
            jax.ShapeDtypeStruct((VOCAB, H), jnp.float32),
            jax.ShapeDtypeStruct((VOCAB, H), jnp.float32),
        ),
    )(t, w0, b0.reshape(1, H), w1, b1.reshape(1, H), wn0)


def _bn_cols(x, g, b):
    m = jnp.mean(x, axis=0, keepdims=True)
    v = jnp.mean((x - m) ** 2, axis=0, keepdims=True)
    return (x - m) / jnp.sqrt(v + 1e-5) * g + b


def _mid_kernel(xe_ref, aa_ref, ws0_ref, b0_ref, g0_ref, bb0_ref,
                wn1_ref, ws1_ref, y1_ref, xs1_ref):
    x = xe_ref[:N]
    h = jnp.dot(x, ws0_ref[...], preferred_element_type=jnp.float32)
    h = h + b0_ref[...] + aa_ref[:N]
    x1 = _leaky(_bn_cols(h, g0_ref[...], bb0_ref[...]))
    y1_ref[...] = jnp.dot(x1, wn1_ref[...], preferred_element_type=jnp.float32)
    xs1_ref[...] = jnp.dot(x1, ws1_ref[...], preferred_element_type=jnp.float32)


def _mid(xe, aa, ws0, b0, g0, bb0, wn1, ws1):
    return pl.pallas_call(
        _mid_kernel,
        out_shape=(
            jax.ShapeDtypeStruct((N, H), jnp.float32),
            jax.ShapeDtypeStruct((N, H), jnp.float32),
        ),
    )(xe, aa, ws0, b0.reshape(1, H), g0.reshape(1, H), bb0.reshape(1, H),
      wn1, ws1)


def _fin_kernel(xs1_ref, aa_ref, b1_ref, g1_ref, bb1_ref, x2_ref):
    h = xs1_ref[...] + b1_ref[...] + aa_ref[:N]
    x2_ref[...] = _bn_cols(h, g1_ref[...], bb1_ref[...])


def _fin(xs1, aa, b1, g1, bb1):
    return pl.pallas_call(
        _fin_kernel,
        out_shape=jax.ShapeDtypeStruct((N, H), jnp.float32),
    )(xs1, aa, b1.reshape(1, H), g1.reshape(1, H), bb1.reshape(1, H))


def _mask_row(src_v, dst_v, i, c):
    for j in range(EC // 16):
        d = dst_v[i, pl.ds(j * 16, 16)] - c * AGH
        ok = (d >= 0) & (d < AGH)
        dst_v[i, pl.ds(j * 16, 16)] = jnp.where(ok, d, -1)
        sv = src_v[i, pl.ds(j * 16, 16)]
        src_v[i, pl.ds(j * 16, 16)] = jnp.where(ok, sv, -1)


def _edge_pass(table, src4, dst4, zer, agg_sh, agg_out,
               src_v, dst_v, bufv, gsem, ssem, c, s):
    bg = SBR // BR

    pltpu.sync_copy(zer, agg_sh.at[pl.ds(s * ZR, ZR)])
    plsc.subcore_barrier()

    def sblk(sb, carry):
        pltpu.sync_copy(src4.at[s, sb], src_v)
        pltpu.sync_copy(dst4.at[s, sb], dst_v)

        def prep(i, c2):
            _mask_row(src_v, dst_v, i, c)
            return c2
        lax.fori_loop(0, SBR, prep, 0)

        prev_scat = []
        for g in range(SBR // BR):
            base = g * BR
            for d in prev_scat:
                d.wait()
            gds = [pltpu.async_copy(
                       table.at[plsc.Indices(src_v.at[base + u], ignored_value=-1)],
                       bufv.at[u], gsem)
                   for u in range(BR)]
            prev_scat = []
            for u in range(BR):
                gds[u].wait()
                prev_scat.append(
                    pltpu.async_copy(
                        bufv.at[u],
                        agg_sh.at[plsc.Indices(dst_v.at[base + u], ignored_value=-1)],
                        ssem, add=True))
        for d in prev_scat:
            d.wait()
        return carry
    lax.fori_loop(0, SB, sblk, 0)

    plsc.subcore_barrier()
    pltpu.sync_copy(agg_sh.at[pl.ds(s * WBT, WBT)],
                    agg_out.at[pl.ds(c * AGH + s * WBT, WBT)])


def _pre_body(tx, ty, f2, xenc, y0, fidx_v, bufa, bufb, sema, semb):
    c = lax.axis_index("c")
    s = lax.axis_index("s")
    wid = c * NS + s

    pltpu.sync_copy(f2.at[wid], fidx_v)
    nk = XPT // XC
    ga = [pltpu.async_copy(tx.at[fidx_v.at[k]], bufa.at[k], sema)
          for k in range(nk)]
    gb = [pltpu.async_copy(ty.at[fidx_v.at[k]], bufb.at[k], semb)
          for k in range(nk)]
    for k in range(nk):
        ga[k].wait()
        pltpu.sync_copy(bufa.at[k], xenc.at[pl.ds(wid * XPT + k * XC, XC)])
        gb[k].wait()
        pltpu.sync_copy(bufb.at[k], y0.at[pl.ds(wid * XPT + k * XC, XC)])


def _pre(tx, ty, f2):
    return pl.kernel(
        _pre_body,
        out_type=(
            jax.ShapeDtypeStruct((NPAD, H), jnp.float32),
            jax.ShapeDtypeStruct((NPAD, H), jnp.float32),
        ),
        mesh=_MESH,
        compiler_params=_SC_PARAMS,
        scratch_types=[
            pltpu.VMEM((XPT // XC, XC), jnp.int32),
            pltpu.VMEM((XPT // XC, XC, H), jnp.float32),
            pltpu.VMEM((XPT // XC, XC, H), jnp.float32),
            pltpu.SemaphoreType.DMA,
            pltpu.SemaphoreType.DMA,
        ],
    )(tx, ty, f2)


def _hop_body(table, src4, dst4, zer, agg,
              src_v, dst_v, bufv, agg_sh, gsem, ssem):
    c = lax.axis_index("c")
    s = lax.axis_index("s")
    _edge_pass(table, src4, dst4, zer, agg_sh, agg,
               src_v, dst_v, bufv, gsem, ssem, c, s)


def _hop(table, src4, dst4, zer):
    return pl.kernel(
        _hop_body,
        out_type=jax.ShapeDtypeStruct((NAGG, H), jnp.float32),
        mesh=_MESH,
        compiler_params=_SC_PARAMS,
        scratch_types=[
            pltpu.VMEM((SBR, EC), jnp.int32),
            pltpu.VMEM((SBR, EC), jnp.int32),
            pltpu.VMEM((BR, EC, H), jnp.float32),
            pltpu.VMEM_SHARED((AGR, H), jnp.float32),
            pltpu.SemaphoreType.DMA,
            pltpu.SemaphoreType.DMA,
        ],
    )(table, src4, dst4, zer)


def _dec_body(x2, e0, e1, labels, pred, pav,
              idx0_v, idx1_v, lab_v, bufa, bufb, pacc_v, out_v, pav_v,
              sema, semb):
    c = lax.axis_index("c")
    s = lax.axis_index("s")
    wid = c * NS + s

    pltpu.sync_copy(e0.at[wid], idx0_v)
    pltpu.sync_copy(e1.at[wid], idx1_v)
    pltpu.sync_copy(labels, lab_v)

    def issue(i, p):
        cpa = pltpu.async_copy(x2.at[idx0_v.at[i]], bufa.at[p], sema)
        cpb = pltpu.async_copy(x2.at[idx1_v.at[i]], bufb.at[p], semb)
        return cpa, cpb

    def drain(i, p):
        pltpu.make_async_copy(x2.at[idx0_v.at[i]], bufa.at[p], sema).wait()
        pltpu.make_async_copy(x2.at[idx1_v.at[i]], bufb.at[p], semb).wait()

    def compute(i, p):
        ba = bufa.at[p]
        bb = bufb.at[p]

        def dot_body(e, carry2):
            acc = ba[e, pl.ds(0, 16)] * bb[e, pl.ds(0, 16)]
            for k in range(1, H // 16):
                acc = acc + ba[e, pl.ds(k * 16, 16)] * bb[e, pl.ds(k * 16, 16)]
            pacc_v[pl.ds(e * 16, 16)] = acc
            return carry2
        lax.fori_loop(0, LC, dot_body, 0)

        lanes = lax.iota(jnp.int32, 16)
        for g in range(LC // 16):
            ebase = (g * 16 + lanes) * 16
            tot = plsc.load_gather(pacc_v, [ebase])
            for l in range(1, 16):
                tot = tot + plsc.load_gather(pacc_v, [ebase + l])
            out_v[pl.ds(g * 16, 16)] = tot

        for j in range(LC // 16):
            vidx = idx1_v[i, pl.ds(j * 16, 16)]
            pav_v[pl.ds(j * 16, 16)] = plsc.load_gather(lab_v, [vidx])

        base = wid * (LRT * LC) + i * LC
        pltpu.sync_copy(out_v, pred.at[pl.ds(base, LC)])
        pltpu.sync_copy(pav_v, pav.at[pl.ds(base, LC)])

    issue(0, 0)

    def kloop(k, carry):
        i0 = 2 * k
        drain(i0, 0)
        issue(i0 + 1, 1)
        compute(i0, 0)
        drain(i0 + 1, 1)

        @pl.when(k < LRT // 2 - 1)
        def _():
            issue(i0 + 2, 0)
        compute(i0 + 1, 1)
        return carry
    lax.fori_loop(0, LRT // 2, kloop, 0)


def _decoder(x2, e0, e1, labels):
    return pl.kernel(
        _dec_body,
        out_type=(
            jax.ShapeDtypeStruct((LPAD,), jnp.float32),
            jax.ShapeDtypeStruct((LPAD,), jnp.float32),
        ),
        mesh=_MESH,
        compiler_params=_SC_PARAMS,
        scratch_types=[
            pltpu.VMEM((LRT, LC), jnp.int32),
            pltpu.VMEM((LRT, LC), jnp.int32),
            pltpu.VMEM((N,), jnp.float32),
            pltpu.VMEM((2, LC, H), jnp.float32),
            pltpu.VMEM((2, LC, H), jnp.float32),
            pltpu.VMEM((LC * 16,), jnp.float32),
            pltpu.VMEM((LC,), jnp.float32),
            pltpu.VMEM((LC,), jnp.float32),
            pltpu.SemaphoreType.DMA,
            pltpu.SemaphoreType.DMA,
        ],
    )(x2, e0, e1, labels)


def kernel(node_feature, edge_index, edge_label_index, node_label, emb_table,
           enc_W0, enc_b0, enc_W1, enc_b1,
           conv0_Wself, conv0_Wnbr, conv0_b,
           conv1_Wself, conv1_Wnbr, conv1_b,
           bn0_g, bn0_b, bn1_g, bn1_b):
    src4 = jnp.concatenate(
        [edge_index[0], jnp.zeros((EPAD - E,), jnp.int32)]).reshape(NS, SB, SBR, EC)
    dst4 = jnp.concatenate(
        [edge_index[1], jnp.full((EPAD - E,), N, jnp.int32)]).reshape(NS, SB, SBR, EC)
    f2 = jnp.concatenate(
        [node_feature, jnp.zeros((NPAD - N,), jnp.int32)]).reshape(NW, XPT // XC, XC)
    e0 = jnp.concatenate(
        [edge_label_index[0], jnp.zeros((LPAD - L,), jnp.int32)]).reshape(NW, LRT, LC)
    e1 = jnp.concatenate(
        [edge_label_index[1], jnp.zeros((LPAD - L,), jnp.int32)]).reshape(NW, LRT, LC)
    zer = jnp.zeros((ZR, H), jnp.float32)

    t_x, t_y0 = _enc_table(emb_table, enc_W0, enc_b0, enc_W1, enc_b1, conv0_Wnbr)
    x_enc, y0 = _pre(t_x, t_y0, f2)
    agg0 = _hop(y0, src4, dst4, zer)
    y1, xs1 = _mid(x_enc, agg0, conv0_Wself, conv0_b, bn0_g, bn0_b,
                   conv1_Wnbr, conv1_Wself)
    agg1 = _hop(y1, src4, dst4, zer)
    x2 = _fin(xs1, agg1, conv1_b, bn1_g, bn1_b)
    pred_pad, pav_pad = _decoder(x2, e0, e1, node_label)
    return (pred_pad[:L], pav_pad[:L])


# speedup vs baseline: 1.4888x; 1.4888x over previous
import functools
import jax
import jax.numpy as jnp
from jax import lax
from jax.experimental import pallas as pl
from jax.experimental.pallas import tpu as pltpu
from jax.experimental.pallas import tpu_sc as plsc

N = 10000
E = 320000
L = 100000
H = 128
VOCAB = 1000

NC = 2
NS = 16
NW = NC * NS

EC = 80
EPAD = E
ERT = EPAD // (NS * EC)
BR = 5
SBR = 25
SB = ERT // SBR
AGH = 5120
AGR = AGH
ZR = AGR // NS
WBT = AGH // NS
NAGG = NC * AGH

NPAD = 10240
XPT = NPAD // NW
XC = 80
LPAD = 100352
LC = 112
LRT = LPAD // (NW * LC)

_MESH = plsc.VectorSubcoreMesh(
    core_axis_name="c", subcore_axis_name="s", num_cores=NC, num_subcores=NS)
_SC_PARAMS = pltpu.CompilerParams(needs_layout_passes=False)


def _leaky(x):
    return jnp.where(x >= 0, x, 0.01 * x)


def _enc_table_kernel(t_ref, w0_ref, b0_ref, w1_ref, b1_ref, wn0_ref,
                      tx_ref, ty_ref):
    x = t_ref[...]
    x = _leaky(jnp.dot(x, w0_ref[...], preferred_element_type=jnp.float32) + b0_ref[...])
    x = _leaky(jnp.dot(x, w1_ref[...], preferred_element_type=jnp.float32) + b1_ref[...])
    tx_ref[...] = x
    ty_ref[...] = jnp.dot(x, wn0_ref[...], preferred_element_type=jnp.float32)


def _enc_table(t, w0, b0, w1, b1, wn0):
    return pl.pallas_call(
        _enc_table_kernel,
        out_shape=(
            jax.ShapeDtypeStruct((VOCAB, H), jnp.float32),
            jax.ShapeDtypeStruct((VOCAB, H), jnp.float32),
        ),
    )(t, w0, b0.reshape(1, H), w1, b1.reshape(1, H), wn0)


def _bn_cols(x, g, b):
    m = jnp.mean(x, axis=0, keepdims=True)
    v = jnp.mean((x - m) ** 2, axis=0, keepdims=True)
    return (x - m) / jnp.sqrt(v + 1e-5) * g + b


def _mid_kernel(xe_ref, aa_ref, ws0_ref, b0_ref, g0_ref, bb0_ref,
                wn1_ref, ws1_ref, y1_ref, xs1_ref):
    x = xe_ref[:N]
    h = jnp.dot(x, ws0_ref[...], preferred_element_type=jnp.float32)
    h = h + b0_ref[...] + aa_ref[:N]
    x1 = _leaky(_bn_cols(h, g0_ref[...], bb0_ref[...]))
    y1_ref[...] = jnp.dot(x1, wn1_ref[...], preferred_element_type=jnp.float32)
    xs1_ref[...] = jnp.dot(x1, ws1_ref[...], preferred_element_type=jnp.float32)


def _mid(xe, aa, ws0, b0, g0, bb0, wn1, ws1):
    return pl.pallas_call(
        _mid_kernel,
        out_shape=(
            jax.ShapeDtypeStruct((N, H), jnp.float32),
            jax.ShapeDtypeStruct((N, H), jnp.float32),
        ),
    )(xe, aa, ws0, b0.reshape(1, H), g0.reshape(1, H), bb0.reshape(1, H),
      wn1, ws1)


def _fin_kernel(xs1_ref, aa_ref, b1_ref, g1_ref, bb1_ref, x2_ref):
    h = xs1_ref[...] + b1_ref[...] + aa_ref[:N]
    x2_ref[...] = _bn_cols(h, g1_ref[...], bb1_ref[...])


def _fin(xs1, aa, b1, g1, bb1):
    return pl.pallas_call(
        _fin_kernel,
        out_shape=jax.ShapeDtypeStruct((N, H), jnp.float32),
    )(xs1, aa, b1.reshape(1, H), g1.reshape(1, H), bb1.reshape(1, H))


def _mask_row(src_v, dst_v, i, c):
    for j in range(EC // 16):
        d = dst_v[i, pl.ds(j * 16, 16)] - c * AGH
        ok = (d >= 0) & (d < AGH)
        dst_v[i, pl.ds(j * 16, 16)] = jnp.where(ok, d, -1)
        sv = src_v[i, pl.ds(j * 16, 16)]
        src_v[i, pl.ds(j * 16, 16)] = jnp.where(ok, sv, -1)


def _edge_pass(table, src4, dst4, zer, agg_sh, agg_out,
               src_v, dst_v, bufv, gsem, ssem, c, s):
    bg = SBR // BR

    pltpu.sync_copy(zer, agg_sh.at[pl.ds(s * ZR, ZR)])
    plsc.subcore_barrier()

    def sblk(sb, carry):
        pltpu.sync_copy(src4.at[s, sb], src_v)
        pltpu.sync_copy(dst4.at[s, sb], dst_v)

        def prep(i, c2):
            _mask_row(src_v, dst_v, i, c)
            return c2
        lax.fori_loop(0, SBR, prep, 0)

        prev_scat = []
        for g in range(SBR // BR):
            base = g * BR
            for d in prev_scat:
                d.wait()
            gds = [pltpu.async_copy(
                       table.at[plsc.Indices(src_v.at[base + u], ignored_value=-1)],
                       bufv.at[u], gsem)
                   for u in range(BR)]
            prev_scat = []
            for u in range(BR):
                gds[u].wait()
                prev_scat.append(
                    pltpu.async_copy(
                        bufv.at[u],
                        agg_sh.at[plsc.Indices(dst_v.at[base + u], ignored_value=-1)],
                        ssem, add=True))
        for d in prev_scat:
            d.wait()
        return carry
    lax.fori_loop(0, SB, sblk, 0)

    plsc.subcore_barrier()
    pltpu.sync_copy(agg_sh.at[pl.ds(s * WBT, WBT)],
                    agg_out.at[pl.ds(c * AGH + s * WBT, WBT)])


def _pre_body(tx, ty, f2, xenc, y0, fidx_v, bufa, bufb, sema, semb):
    c = lax.axis_index("c")
    s = lax.axis_index("s")
    wid = c * NS + s

    pltpu.sync_copy(f2.at[wid], fidx_v)
    nk = XPT // XC
    ga = [pltpu.async_copy(tx.at[fidx_v.at[k]], bufa.at[k], sema)
          for k in range(nk)]
    gb = [pltpu.async_copy(ty.at[fidx_v.at[k]], bufb.at[k], semb)
          for k in range(nk)]
    for k in range(nk):
        ga[k].wait()
        pltpu.sync_copy(bufa.at[k], xenc.at[pl.ds(wid * XPT + k * XC, XC)])
        gb[k].wait()
        pltpu.sync_copy(bufb.at[k], y0.at[pl.ds(wid * XPT + k * XC, XC)])


def _pre(tx, ty, f2):
    return pl.kernel(
        _pre_body,
        out_type=(
            jax.ShapeDtypeStruct((NPAD, H), jnp.float32),
            jax.ShapeDtypeStruct((NPAD, H), jnp.float32),
        ),
        mesh=_MESH,
        compiler_params=_SC_PARAMS,
        scratch_types=[
            pltpu.VMEM((XPT // XC, XC), jnp.int32),
            pltpu.VMEM((XPT // XC, XC, H), jnp.float32),
            pltpu.VMEM((XPT // XC, XC, H), jnp.float32),
            pltpu.SemaphoreType.DMA,
            pltpu.SemaphoreType.DMA,
        ],
    )(tx, ty, f2)


def _hop_body(table, src4, dst4, zer, agg,
              src_v, dst_v, bufv, agg_sh, gsem, ssem):
    c = lax.axis_index("c")
    s = lax.axis_index("s")
    _edge_pass(table, src4, dst4, zer, agg_sh, agg,
               src_v, dst_v, bufv, gsem, ssem, c, s)


def _hop(table, src4, dst4, zer):
    return pl.kernel(
        _hop_body,
        out_type=jax.ShapeDtypeStruct((NAGG, H), jnp.float32),
        mesh=_MESH,
        compiler_params=_SC_PARAMS,
        scratch_types=[
            pltpu.VMEM((SBR, EC), jnp.int32),
            pltpu.VMEM((SBR, EC), jnp.int32),
            pltpu.VMEM((BR, EC, H), jnp.float32),
            pltpu.VMEM_SHARED((AGR, H), jnp.float32),
            pltpu.SemaphoreType.DMA,
            pltpu.SemaphoreType.DMA,
        ],
    )(table, src4, dst4, zer)


def _dec_body(x2, e0, e1, labels, pred, pav,
              idx0_v, idx1_v, lab_v, bufa, bufb, pacc_v, out_v, pav_v,
              sema, semb):
    c = lax.axis_index("c")
    s = lax.axis_index("s")
    wid = c * NS + s

    pltpu.sync_copy(e0.at[wid], idx0_v)
    pltpu.sync_copy(e1.at[wid], idx1_v)
    pltpu.sync_copy(labels, lab_v)

    def issue(i, p):
        cpa = pltpu.async_copy(x2.at[idx0_v.at[i]], bufa.at[p], sema)
        cpb = pltpu.async_copy(x2.at[idx1_v.at[i]], bufb.at[p], semb)
        return cpa, cpb

    def drain(i, p):
        pltpu.make_async_copy(x2.at[idx0_v.at[i]], bufa.at[p], sema).wait()
        pltpu.make_async_copy(x2.at[idx1_v.at[i]], bufb.at[p], semb).wait()

    def compute(i, p):
        ba = bufa.at[p]
        bb = bufb.at[p]

        def dot_body(e, carry2):
            acc = ba[e, pl.ds(0, 16)] * bb[e, pl.ds(0, 16)]
            for k in range(1, H // 16):
                acc = acc + ba[e, pl.ds(k * 16, 16)] * bb[e, pl.ds(k * 16, 16)]
            pacc_v[pl.ds(e * 16, 16)] = acc
            return carry2
        lax.fori_loop(0, LC, dot_body, 0)

        lanes = lax.iota(jnp.int32, 16)
        for g in range(LC // 16):
            ebase = (g * 16 + lanes) * 16
            tot = plsc.load_gather(pacc_v, [ebase])
            for l in range(1, 16):
                tot = tot + plsc.load_gather(pacc_v, [ebase + l])
            out_v[pl.ds(g * 16, 16)] = tot

        for j in range(LC // 16):
            vidx = idx1_v[i, pl.ds(j * 16, 16)]
            pav_v[pl.ds(j * 16, 16)] = plsc.load_gather(lab_v, [vidx])

        base = wid * (LRT * LC) + i * LC
        pltpu.sync_copy(out_v, pred.at[pl.ds(base, LC)])
        pltpu.sync_copy(pav_v, pav.at[pl.ds(base, LC)])

    issue(0, 0)

    def kloop(k, carry):
        i0 = 2 * k
        drain(i0, 0)
        issue(i0 + 1, 1)
        compute(i0, 0)
        drain(i0 + 1, 1)

        @pl.when(k < LRT // 2 - 1)
        def _():
            issue(i0 + 2, 0)
        compute(i0 + 1, 1)
        return carry
    lax.fori_loop(0, LRT // 2, kloop, 0)


def _decoder(x2, e0, e1, labels):
    return pl.kernel(
        _dec_body,
        out_type=(
            jax.ShapeDtypeStruct((LPAD,), jnp.float32),
            jax.ShapeDtypeStruct((LPAD,), jnp.float32),
        ),
        mesh=_MESH,
        compiler_params=_SC_PARAMS,
        scratch_types=[
            pltpu.VMEM((LRT, LC), jnp.int32),
            pltpu.VMEM((LRT, LC), jnp.int32),
            pltpu.VMEM((N,), jnp.float32),
            pltpu.VMEM((2, LC, H), jnp.float32),
            pltpu.VMEM((2, LC, H), jnp.float32),
            pltpu.VMEM((LC * 16,), jnp.float32),
            pltpu.VMEM((LC,), jnp.float32),
            pltpu.VMEM((LC,), jnp.float32),
            pltpu.SemaphoreType.DMA,
            pltpu.SemaphoreType.DMA,
        ],
    )(x2, e0, e1, labels)


def kernel(node_feature, edge_index, edge_label_index, node_label, emb_table,
           enc_W0, enc_b0, enc_W1, enc_b1,
           conv0_Wself, conv0_Wnbr, conv0_b,
           conv1_Wself, conv1_Wnbr, conv1_b,
           bn0_g, bn0_b, bn1_g, bn1_b):
    src4 = edge_index[0].reshape(NS, SB, SBR, EC)
    dst4 = edge_index[1].reshape(NS, SB, SBR, EC)
    f2 = jnp.concatenate(
        [node_feature, jnp.zeros((NPAD - N,), jnp.int32)]).reshape(NW, XPT // XC, XC)
    e0 = jnp.concatenate(
        [edge_label_index[0], jnp.zeros((LPAD - L,), jnp.int32)]).reshape(NW, LRT, LC)
    e1 = jnp.concatenate(
        [edge_label_index[1], jnp.zeros((LPAD - L,), jnp.int32)]).reshape(NW, LRT, LC)
    zer = jnp.zeros((ZR, H), jnp.float32)

    t_x, t_y0 = _enc_table(emb_table, enc_W0, enc_b0, enc_W1, enc_b1, conv0_Wnbr)
    x_enc, y0 = _pre(t_x, t_y0, f2)
    agg0 = _hop(y0, src4, dst4, zer)
    y1, xs1 = _mid(x_enc, agg0, conv0_Wself, conv0_b, bn0_g, bn0_b,
                   conv1_Wnbr, conv1_Wself)
    agg1 = _hop(y1, src4, dst4, zer)
    x2 = _fin(xs1, agg1, conv1_b, bn1_g, bn1_b)
    pred_pad, pav_pad = _decoder(x2, e0, e1, node_label)
    return (pred_pad[:L], pav_pad[:L])
